# Initial kernel scaffold; baseline (speedup 1.0000x reference)
import jax
import jax.numpy as jnp
from jax.experimental import pallas as pl


def kernel(phoneme_tensor, onset_table, rhyme_table, tone_table):
    raise NotImplementedError("write your pallas kernel here")


# speedup vs baseline: 2.8423x; 2.8423x over previous
import functools
import math

import jax
import jax.numpy as jnp
from jax import lax
from jax.experimental import pallas as pl
from jax.experimental.pallas import tpu as pltpu
from jax.experimental.pallas import tpu_sc as plsc

NC = 2
NS = 16
L = 16
NW = NC * NS

VOCAB = 1000
D = 128
NUM_TABLES = 3
SCALE = math.sqrt(128.0)

CHUNK = 128


def _body(ph, onset, rhyme, tone, out, idxraw_v, idx0_v, idx1_v, idx2_v,
          rows0_v, rows1_v, rows2_v, sem):
    cid = lax.axis_index("c")
    sid = lax.axis_index("s")
    wid = sid * NC + cid

    n_tokens = out.shape[0]
    per_w = n_tokens // NW
    n_chunks = per_w // CHUNK
    base = wid * per_w
    tables = (onset, rhyme, tone)
    idx_refs = (idx0_v, idx1_v, idx2_v)
    row_refs = (rows0_v, rows1_v, rows2_v)

    def chunk_body(c, _):
        tok0 = base + c * CHUNK
        pltpu.sync_copy(ph.at[pl.ds(tok0 * NUM_TABLES, CHUNK * NUM_TABLES)],
                        idxraw_v)
        for g in range(CHUNK // L):
            rid = (lax.iota(jnp.int32, 16) + g * L) * NUM_TABLES
            for j in range(NUM_TABLES):
                idx_refs[j][pl.ds(g * L, L)] = plsc.load_gather(
                    idxraw_v, [rid + j])
        handles = [
            pltpu.async_copy(tables[j].at[idx_refs[j]], row_refs[j], sem)
            for j in range(NUM_TABLES)
        ]
        for h in handles:
            h.wait()

        def scale_row(i, _):
            for j in range(NUM_TABLES):
                for h in range(D // L):
                    row_refs[j][i, pl.ds(h * L, L)] = (
                        row_refs[j][i, pl.ds(h * L, L)] * SCALE)
            return 0

        lax.fori_loop(0, CHUNK, scale_row, 0)
        for j in range(NUM_TABLES):
            pltpu.sync_copy(
                row_refs[j],
                out.at[pl.ds(tok0, CHUNK), pl.ds(j * D, D)],
            )
        return 0

    lax.fori_loop(0, n_chunks, chunk_body, 0)


def _make(n_tokens):
    mesh = plsc.VectorSubcoreMesh(core_axis_name="c", subcore_axis_name="s")
    return pl.kernel(
        _body,
        out_type=jax.ShapeDtypeStruct((n_tokens, NUM_TABLES * D), jnp.float32),
        mesh=mesh,
        compiler_params=pltpu.CompilerParams(needs_layout_passes=False),
        scratch_types=[
            pltpu.VMEM((CHUNK * NUM_TABLES,), jnp.int32),
            pltpu.VMEM((CHUNK,), jnp.int32),
            pltpu.VMEM((CHUNK,), jnp.int32),
            pltpu.VMEM((CHUNK,), jnp.int32),
            pltpu.VMEM((CHUNK, D), jnp.float32),
            pltpu.VMEM((CHUNK, D), jnp.float32),
            pltpu.VMEM((CHUNK, D), jnp.float32),
            pltpu.SemaphoreType.DMA,
        ],
    )


@jax.jit
def kernel(phoneme_tensor, onset_table, rhyme_table, tone_table):
    b, s, _ = phoneme_tensor.shape
    ph = phoneme_tensor.reshape(b * s * NUM_TABLES).astype(jnp.int32)
    out = _make(b * s)(ph, onset_table, rhyme_table, tone_table)
    return out.reshape(b, s, NUM_TABLES * D)


# speedup vs baseline: 3.2903x; 1.1576x over previous
import functools
import math

import jax
import jax.numpy as jnp
from jax import lax
from jax.experimental import pallas as pl
from jax.experimental.pallas import tpu as pltpu
from jax.experimental.pallas import tpu_sc as plsc

NC = 2
NS = 16
L = 16
NW = NC * NS

VOCAB = 1000
D = 128
NUM_TABLES = 3
SCALE = math.sqrt(128.0)

CHUNK = 128
STAGE_ROWS = 64


def _body(ph, onset, rhyme, tone, out, shared, stage_v, idxraw_v,
          idx0_v, idx1_v, idx2_v, rows0_v, rows1_v, rows2_v, sem):
    cid = lax.axis_index("c")
    sid = lax.axis_index("s")
    wid = sid * NC + cid

    r0 = jnp.minimum(sid * STAGE_ROWS, VOCAB - STAGE_ROWS)
    for j, tab in enumerate((onset, rhyme, tone)):
        pltpu.sync_copy(tab.at[pl.ds(r0, STAGE_ROWS)], stage_v)

        def scale_row(i, _):
            for h in range(D // L):
                stage_v[i, pl.ds(h * L, L)] = stage_v[i, pl.ds(h * L, L)] * SCALE
            return 0

        lax.fori_loop(0, STAGE_ROWS, scale_row, 0)
        pltpu.sync_copy(stage_v, shared.at[pl.ds(j * VOCAB + r0, STAGE_ROWS)])
    plsc.subcore_barrier()

    n_tokens = out.shape[0]
    per_w = n_tokens // NW
    n_chunks = per_w // CHUNK
    base = wid * per_w
    idx_refs = (idx0_v, idx1_v, idx2_v)
    row_refs = (rows0_v, rows1_v, rows2_v)

    def chunk_body(c, _):
        tok0 = base + c * CHUNK
        pltpu.sync_copy(ph.at[pl.ds(tok0 * NUM_TABLES, CHUNK * NUM_TABLES)],
                        idxraw_v)
        for g in range(CHUNK // L):
            rid = (lax.iota(jnp.int32, 16) + g * L) * NUM_TABLES
            for j in range(NUM_TABLES):
                idx_refs[j][pl.ds(g * L, L)] = (
                    plsc.load_gather(idxraw_v, [rid + j]) + j * VOCAB)
        handles = [
            pltpu.async_copy(shared.at[idx_refs[j]], row_refs[j], sem)
            for j in range(NUM_TABLES)
        ]
        for h in handles:
            h.wait()
        for j in range(NUM_TABLES):
            pltpu.sync_copy(
                row_refs[j],
                out.at[pl.ds(tok0, CHUNK), pl.ds(j * D, D)],
            )
        return 0

    lax.fori_loop(0, n_chunks, chunk_body, 0)


def _make(n_tokens):
    mesh = plsc.VectorSubcoreMesh(core_axis_name="c", subcore_axis_name="s")
    return pl.kernel(
        _body,
        out_type=jax.ShapeDtypeStruct((n_tokens, NUM_TABLES * D), jnp.float32),
        mesh=mesh,
        compiler_params=pltpu.CompilerParams(needs_layout_passes=False),
        scratch_types=[
            pltpu.VMEM_SHARED((NUM_TABLES * VOCAB, D), jnp.float32),
            pltpu.VMEM((STAGE_ROWS, D), jnp.float32),
            pltpu.VMEM((CHUNK * NUM_TABLES,), jnp.int32),
            pltpu.VMEM((CHUNK,), jnp.int32),
            pltpu.VMEM((CHUNK,), jnp.int32),
            pltpu.VMEM((CHUNK,), jnp.int32),
            pltpu.VMEM((CHUNK, D), jnp.float32),
            pltpu.VMEM((CHUNK, D), jnp.float32),
            pltpu.VMEM((CHUNK, D), jnp.float32),
            pltpu.SemaphoreType.DMA,
        ],
    )


@jax.jit
def kernel(phoneme_tensor, onset_table, rhyme_table, tone_table):
    b, s, _ = phoneme_tensor.shape
    ph = phoneme_tensor.reshape(b * s * NUM_TABLES).astype(jnp.int32)
    out = _make(b * s)(ph, onset_table, rhyme_table, tone_table)
    return out.reshape(b, s, NUM_TABLES * D)


# speedup vs baseline: 3.7295x; 1.1335x over previous
import functools
import math

import jax
import jax.numpy as jnp
from jax import lax
from jax.experimental import pallas as pl
from jax.experimental.pallas import tpu as pltpu
from jax.experimental.pallas import tpu_sc as plsc

NC = 2
NS = 16
L = 16
NW = NC * NS

VOCAB = 1000
D = 128
NUM_TABLES = 3
SCALE = math.sqrt(128.0)

CHUNK = 128
STAGE_ROWS = 64
NSLOTS = 2


def _body(ph, onset, rhyme, tone, out, shared,
          idxraw0, idxraw1, i00, i01, i02, i10, i11, i12,
          r00, r01, r02, r10, r11, r12, sg0, sg1, sw0, sw1):
    cid = lax.axis_index("c")
    sid = lax.axis_index("s")
    wid = sid * NC + cid

    idxraw = (idxraw0, idxraw1)
    idx = ((i00, i01, i02), (i10, i11, i12))
    rows = ((r00, r01, r02), (r10, r11, r12))
    semg = (sg0, sg1)
    semw = (sw0, sw1)

    r0 = jnp.minimum(sid * STAGE_ROWS, VOCAB - STAGE_ROWS)
    for j, tab in enumerate((onset, rhyme, tone)):
        pltpu.sync_copy(tab.at[pl.ds(r0, STAGE_ROWS)], r00.at[pl.ds(0, STAGE_ROWS)])

        def scale_row(i, _):
            for h in range(D // L):
                r00[i, pl.ds(h * L, L)] = r00[i, pl.ds(h * L, L)] * SCALE
            return 0

        lax.fori_loop(0, STAGE_ROWS, scale_row, 0)
        pltpu.sync_copy(r00.at[pl.ds(0, STAGE_ROWS)],
                        shared.at[pl.ds(j * VOCAB + r0, STAGE_ROWS)])
    plsc.subcore_barrier()

    n_tokens = out.shape[0]
    per_w = n_tokens // NW
    n_chunks = per_w // CHUNK
    base = wid * per_w

    def load_idx(c, s):
        tok0 = base + c * CHUNK
        pltpu.sync_copy(ph.at[pl.ds(tok0 * NUM_TABLES, CHUNK * NUM_TABLES)],
                        idxraw[s])
        for g in range(CHUNK // L):
            rid = (lax.iota(jnp.int32, 16) + g * L) * NUM_TABLES
            for j in range(NUM_TABLES):
                idx[s][j][pl.ds(g * L, L)] = (
                    plsc.load_gather(idxraw[s], [rid + j]) + j * VOCAB)

    def fire_gathers(s):
        for j in range(NUM_TABLES):
            pltpu.async_copy(shared.at[idx[s][j]], rows[s][j], semg[s])

    def wait_gathers(s):
        for j in range(NUM_TABLES):
            pltpu.make_async_copy(shared.at[idx[s][j]], rows[s][j],
                                  semg[s]).wait()

    def out_slice(c, j):
        return out.at[pl.ds(base + c * CHUNK, CHUNK), pl.ds(j * D, D)]

    for s in range(NSLOTS):
        load_idx(jnp.int32(s), s)
        fire_gathers(s)

    def pair_body(i, _):
        for s in range(NSLOTS):
            c = i * NSLOTS + s
            wait_gathers(s)
            for j in range(NUM_TABLES):
                pltpu.async_copy(rows[s][j], out_slice(c, j), semw[s])
            c2 = jnp.where(c + NSLOTS < n_chunks, c + NSLOTS, 0)
            load_idx(c2, s)
            for j in range(NUM_TABLES):
                pltpu.make_async_copy(rows[s][j], out_slice(c, j),
                                      semw[s]).wait()
            fire_gathers(s)
        return 0

    lax.fori_loop(0, n_chunks // NSLOTS, pair_body, 0)
    for s in range(NSLOTS):
        wait_gathers(s)


def _make(n_tokens):
    mesh = plsc.VectorSubcoreMesh(core_axis_name="c", subcore_axis_name="s")
    return pl.kernel(
        _body,
        out_type=jax.ShapeDtypeStruct((n_tokens, NUM_TABLES * D), jnp.float32),
        mesh=mesh,
        compiler_params=pltpu.CompilerParams(needs_layout_passes=False),
        scratch_types=[
            pltpu.VMEM_SHARED((NUM_TABLES * VOCAB, D), jnp.float32),
            pltpu.VMEM((CHUNK * NUM_TABLES,), jnp.int32),
            pltpu.VMEM((CHUNK * NUM_TABLES,), jnp.int32),
            pltpu.VMEM((CHUNK,), jnp.int32),
            pltpu.VMEM((CHUNK,), jnp.int32),
            pltpu.VMEM((CHUNK,), jnp.int32),
            pltpu.VMEM((CHUNK,), jnp.int32),
            pltpu.VMEM((CHUNK,), jnp.int32),
            pltpu.VMEM((CHUNK,), jnp.int32),
            pltpu.VMEM((CHUNK, D), jnp.float32),
            pltpu.VMEM((CHUNK, D), jnp.float32),
            pltpu.VMEM((CHUNK, D), jnp.float32),
            pltpu.VMEM((CHUNK, D), jnp.float32),
            pltpu.VMEM((CHUNK, D), jnp.float32),
            pltpu.VMEM((CHUNK, D), jnp.float32),
            pltpu.SemaphoreType.DMA,
            pltpu.SemaphoreType.DMA,
            pltpu.SemaphoreType.DMA,
            pltpu.SemaphoreType.DMA,
        ],
    )


@jax.jit
def kernel(phoneme_tensor, onset_table, rhyme_table, tone_table):
    b, s, _ = phoneme_tensor.shape
    ph = phoneme_tensor.reshape(b * s * NUM_TABLES).astype(jnp.int32)
    out = _make(b * s)(ph, onset_table, rhyme_table, tone_table)
    return out.reshape(b, s, NUM_TABLES * D)


# speedup vs baseline: 5.7918x; 1.5530x over previous
import functools
import math

import jax
import jax.numpy as jnp
from jax import lax
from jax.experimental import pallas as pl
from jax.experimental.pallas import tpu as pltpu
from jax.experimental.pallas import tpu_sc as plsc

NC = 2
NS = 16
L = 16
NW = NC * NS

VOCAB = 1000
D = 128
NUM_TABLES = 3
SCALE = math.sqrt(128.0)

SEQ = 50
SEQ_PAD = 56
NB = 1
NSLOTS = 2
IDX_BLOCK = 400
STAGE_ROWS = 32


def _body(ph, onset, rhyme, tone, out, shared, stage_v, idxraw_v,
          i0, i1, i2,
          r00, r01, r02, r10, r11, r12,
          sg0, sg1, sw0, sw1):
    cid = lax.axis_index("c")
    sid = lax.axis_index("s")
    wid = sid * NC + cid

    n_batch = out.shape[0]
    rows_per_w = n_batch // NW
    tok_per_w = rows_per_w * SEQ
    n_steps = rows_per_w // NB
    row0 = wid * rows_per_w

    idx = (i0, i1, i2)
    rows = ((r00, r01, r02), (r10, r11, r12))
    semg = (sg0, sg1)
    semw = (sw0, sw1)

    for j, tab in enumerate((onset, rhyme, tone)):
        for k in range(2):
            r0 = jnp.minimum(sid * (2 * STAGE_ROWS), VOCAB - 2 * STAGE_ROWS) \
                + k * STAGE_ROWS
            pltpu.sync_copy(tab.at[pl.ds(r0, STAGE_ROWS)], stage_v)

            def scale_row(i, _):
                for h in range(D // L):
                    stage_v[i, pl.ds(h * L, L)] = (
                        stage_v[i, pl.ds(h * L, L)] * SCALE)
                return 0

            lax.fori_loop(0, STAGE_ROWS, scale_row, 0)
            pltpu.sync_copy(stage_v,
                            shared.at[pl.ds(j * VOCAB + r0, STAGE_ROWS)])
    plsc.subcore_barrier()

    n_blocks = tok_per_w // IDX_BLOCK

    def deint_block(blk, _):
        t0 = wid * tok_per_w + blk * IDX_BLOCK
        pltpu.sync_copy(
            ph.at[pl.ds(t0 * NUM_TABLES, IDX_BLOCK * NUM_TABLES)], idxraw_v)
        for g in range(IDX_BLOCK // L):
            tloc = lax.iota(jnp.int32, 16) + g * L
            tw = tloc + blk * IDX_BLOCK
            pos = tw + (SEQ_PAD - SEQ) * (tw // SEQ)
            for j in range(NUM_TABLES):
                v = plsc.load_gather(idxraw_v, [tloc * NUM_TABLES + j])
                plsc.store_scatter(idx[j], [pos], v + j * VOCAB)
        return 0

    lax.fori_loop(0, n_blocks, deint_block, 0)

    def fire_gathers(s, step):
        for j in range(NUM_TABLES):
            pltpu.async_copy(
                shared.at[idx[j].at[pl.ds(step * SEQ_PAD, SEQ)]],
                rows[s][j].at[0], semg[s])

    def wait_gathers(s, step):
        for j in range(NUM_TABLES):
            pltpu.make_async_copy(
                shared.at[idx[j].at[pl.ds(step * SEQ_PAD, SEQ)]],
                rows[s][j].at[0], semg[s]).wait()

    def out_slice(step, j):
        return out.at[pl.ds(row0 + step * NB, NB), :, pl.ds(j * D, D)]

    for s in range(NSLOTS):
        fire_gathers(s, jnp.int32(s))

    def pair_body(i, _):
        for s in range(NSLOTS):
            step = i * NSLOTS + s
            wait_gathers(s, step)
            for j in range(NUM_TABLES):
                pltpu.async_copy(rows[s][j], out_slice(step, j), semw[s])
            for j in range(NUM_TABLES):
                pltpu.make_async_copy(rows[s][j], out_slice(step, j),
                                      semw[s]).wait()
            step2 = jnp.where(step + NSLOTS < n_steps, step + NSLOTS, 0)
            fire_gathers(s, step2)
        return 0

    lax.fori_loop(0, n_steps // NSLOTS, pair_body, 0)
    for s in range(NSLOTS):
        wait_gathers(s, jnp.int32(0))


def _make(n_batch, seq):
    mesh = plsc.VectorSubcoreMesh(core_axis_name="c", subcore_axis_name="s")
    return pl.kernel(
        _body,
        out_type=jax.ShapeDtypeStruct((n_batch, seq, NUM_TABLES * D),
                                      jnp.float32),
        mesh=mesh,
        compiler_params=pltpu.CompilerParams(needs_layout_passes=False),
        scratch_types=[
            pltpu.VMEM_SHARED((NUM_TABLES * VOCAB, D), jnp.float32),
            pltpu.VMEM((STAGE_ROWS, D), jnp.float32),
            pltpu.VMEM((IDX_BLOCK * NUM_TABLES,), jnp.int32),
            pltpu.VMEM((4096 // NW * SEQ_PAD,), jnp.int32),
            pltpu.VMEM((4096 // NW * SEQ_PAD,), jnp.int32),
            pltpu.VMEM((4096 // NW * SEQ_PAD,), jnp.int32),
            pltpu.VMEM((NB, SEQ, D), jnp.float32),
            pltpu.VMEM((NB, SEQ, D), jnp.float32),
            pltpu.VMEM((NB, SEQ, D), jnp.float32),
            pltpu.VMEM((NB, SEQ, D), jnp.float32),
            pltpu.VMEM((NB, SEQ, D), jnp.float32),
            pltpu.VMEM((NB, SEQ, D), jnp.float32),
            pltpu.SemaphoreType.DMA,
            pltpu.SemaphoreType.DMA,
            pltpu.SemaphoreType.DMA,
            pltpu.SemaphoreType.DMA,
        ],
    )


@jax.jit
def kernel(phoneme_tensor, onset_table, rhyme_table, tone_table):
    b, s, _ = phoneme_tensor.shape
    ph = phoneme_tensor.reshape(b * s * NUM_TABLES).astype(jnp.int32)
    return _make(b, s)(ph, onset_table, rhyme_table, tone_table)


# speedup vs baseline: 20.5675x; 3.5511x over previous
import functools
import math

import jax
import jax.numpy as jnp
from jax import lax
from jax.experimental import pallas as pl
from jax.experimental.pallas import tpu as pltpu
from jax.experimental.pallas import tpu_sc as plsc

NC = 2
NS = 16
L = 16
NW = NC * NS

VOCAB = 1000
D = 128
NUM_TABLES = 3
SCALE = math.sqrt(128.0)

BBLK = 128
STAGE_ROWS = 32


def _body(ph_t, onset, rhyme, tone, out, sh0, sh1, sh2, stage_v,
          ix0, ix1, ix2, r0v, r1v, r2v, sg0, sg1, sg2, sw0, sw1, sw2):
    cid = lax.axis_index("c")
    sid = lax.axis_index("s")
    wid = sid * NC + cid

    seq = out.shape[0]
    b0 = wid * BBLK
    shared = (sh0, sh1, sh2)
    idx = (ix0, ix1, ix2)
    rows = (r0v, r1v, r2v)
    semg = (sg0, sg1, sg2)
    semw = (sw0, sw1, sw2)

    for j, tab in enumerate((onset, rhyme, tone)):
        for k in range(2):
            t0 = jnp.minimum(sid * (2 * STAGE_ROWS), VOCAB - 2 * STAGE_ROWS) \
                + k * STAGE_ROWS
            pltpu.sync_copy(tab.at[pl.ds(t0, STAGE_ROWS)], stage_v)

            def scale_row(i, _):
                for h in range(D // L):
                    stage_v[i, pl.ds(h * L, L)] = (
                        stage_v[i, pl.ds(h * L, L)] * SCALE)
                return 0

            lax.fori_loop(0, STAGE_ROWS, scale_row, 0)
            pltpu.sync_copy(stage_v, shared[j].at[pl.ds(t0, STAGE_ROWS)])
    plsc.subcore_barrier()

    def load_idx(j, s):
        pltpu.sync_copy(ph_t.at[j, s, pl.ds(b0, BBLK)], idx[j])

    def fire_gather(j):
        pltpu.async_copy(shared[j].at[idx[j]], rows[j], semg[j])

    def wait_gather(j):
        pltpu.make_async_copy(shared[j].at[idx[j]], rows[j], semg[j]).wait()

    def out_slice(j, s):
        return out.at[s, pl.ds(b0, BBLK), pl.ds(j * D, D)]

    for j in range(NUM_TABLES):
        load_idx(j, jnp.int32(0))
        fire_gather(j)

    def step(s, _):
        s2 = jnp.where(s + 1 < seq, s + 1, 0)
        for j in range(NUM_TABLES):
            wait_gather(j)
            pltpu.async_copy(rows[j], out_slice(j, s), semw[j])
            load_idx(j, s2)
            pltpu.make_async_copy(rows[j], out_slice(j, s), semw[j]).wait()
            fire_gather(j)
        return 0

    lax.fori_loop(0, seq, step, 0)
    for j in range(NUM_TABLES):
        wait_gather(j)


def _make(n_batch, seq):
    mesh = plsc.VectorSubcoreMesh(core_axis_name="c", subcore_axis_name="s")
    return pl.kernel(
        _body,
        out_type=jax.ShapeDtypeStruct((seq, n_batch, NUM_TABLES * D),
                                      jnp.float32),
        mesh=mesh,
        compiler_params=pltpu.CompilerParams(needs_layout_passes=False),
        scratch_types=[
            pltpu.VMEM_SHARED((VOCAB, D), jnp.float32),
            pltpu.VMEM_SHARED((VOCAB, D), jnp.float32),
            pltpu.VMEM_SHARED((VOCAB, D), jnp.float32),
            pltpu.VMEM((STAGE_ROWS, D), jnp.float32),
            pltpu.VMEM((BBLK,), jnp.int32),
            pltpu.VMEM((BBLK,), jnp.int32),
            pltpu.VMEM((BBLK,), jnp.int32),
            pltpu.VMEM((BBLK, D), jnp.float32),
            pltpu.VMEM((BBLK, D), jnp.float32),
            pltpu.VMEM((BBLK, D), jnp.float32),
            pltpu.SemaphoreType.DMA,
            pltpu.SemaphoreType.DMA,
            pltpu.SemaphoreType.DMA,
            pltpu.SemaphoreType.DMA,
            pltpu.SemaphoreType.DMA,
            pltpu.SemaphoreType.DMA,
        ],
    )


@jax.jit
def kernel(phoneme_tensor, onset_table, rhyme_table, tone_table):
    b, s, _ = phoneme_tensor.shape
    ph_t = phoneme_tensor.astype(jnp.int32).transpose(2, 1, 0)
    out3 = _make(b, s)(ph_t, onset_table, rhyme_table, tone_table)
    return out3.transpose(1, 0, 2)
